# baseline (speedup 1.0000x reference)
import functools

import jax
import jax.numpy as jnp
from jax import lax
from jax.experimental import pallas as pl
from jax.experimental.pallas import tpu as pltpu
from jax.experimental.pallas import tpu_sc as plsc

N_NODES = 100000
D = 128
NUM_GRAPHS = 512

NC = 2
NS = 16
NW = NC * NS

CHUNK = 80
NFULL = N_NODES // CHUNK
TAIL = N_NODES - NFULL * CHUNK
ROWS = 640
RPT = ROWS // NS
NBUF = 4

GH = 32
GL = 16
CNT_BLK = 12800
N_PAD = 102400


def _sc_segment_sum(x3d, batch2d, zsum):
    mesh = plsc.VectorSubcoreMesh(core_axis_name="c", subcore_axis_name="s")

    @functools.partial(
        pl.kernel,
        out_type=jax.ShapeDtypeStruct((NC, ROWS, D), jnp.float32),
        mesh=mesh,
        scratch_types=(
            [pltpu.VMEM((CHUNK,), jnp.int32)] * NBUF
            + [pltpu.VMEM((CHUNK, D), jnp.float32)] * NBUF
            + [pltpu.VMEM_SHARED((ROWS, D), jnp.float32)]
            + [pltpu.SemaphoreType.DMA] * (2 * NBUF)
        ),
    )
    def seg_sum(x_hbm, b_hbm, zsum_hbm, sums_out, *scr):
        cid = lax.axis_index("c")
        sid = lax.axis_index("s")
        wid = sid * NC + cid
        idx_bufs = scr[:NBUF]
        row_bufs = scr[NBUF:2 * NBUF]
        ssum = scr[2 * NBUF]
        sems = scr[2 * NBUF + 1:3 * NBUF + 1]
        ssems = scr[3 * NBUF + 1:]

        pltpu.sync_copy(zsum_hbm.at[pl.ds(RPT * sid, RPT)],
                        ssum.at[pl.ds(RPT * sid, RPT)])
        plsc.subcore_barrier()

        nc_mine = jnp.where(wid < NFULL % NW, NFULL // NW + 1, NFULL // NW)

        def do_slot(j, b):
            bn = (b + 1) % NBUF
            c = wid + NW * j
            pltpu.make_async_copy(b_hbm.at[c], idx_bufs[b], sems[b]).wait()
            pltpu.make_async_copy(x_hbm.at[c], row_bufs[b], sems[b]).wait()
            pltpu.async_copy(row_bufs[b], ssum.at[idx_bufs[b]], ssems[b],
                             add=True)

            @pl.when(j + 1 >= NBUF)
            def _():
                pltpu.make_async_copy(row_bufs[bn], ssum.at[idx_bufs[bn]],
                                      ssems[bn]).wait()

            @pl.when(j + 1 < nc_mine)
            def _():
                c2 = wid + NW * (j + 1)
                pltpu.async_copy(b_hbm.at[c2], idx_bufs[bn], sems[bn])
                pltpu.async_copy(x_hbm.at[c2], row_bufs[bn], sems[bn])

        pltpu.async_copy(b_hbm.at[wid], idx_bufs[0], sems[0])
        pltpu.async_copy(x_hbm.at[wid], row_bufs[0], sems[0])

        def body(g, carry):
            for b in range(NBUF):
                j = NBUF * g + b

                @pl.when(j < nc_mine)
                def _():
                    do_slot(j, b)

            return carry

        lax.fori_loop(0, (NFULL // NW + NBUF) // NBUF, body, 0)

        for b in range(NBUF):
            @pl.when(b != nc_mine % NBUF)
            def _():
                pltpu.make_async_copy(row_bufs[b], ssum.at[idx_bufs[b]],
                                      ssems[b]).wait()

        plsc.subcore_barrier()

        pltpu.sync_copy(ssum.at[pl.ds(RPT * sid, RPT)],
                        sums_out.at[cid, pl.ds(RPT * sid, RPT)])

    return seg_sum(x3d, batch2d, zsum)


def _counts_body(b_ref, out_ref):
    i = pl.program_id(0)

    @pl.when(i == 0)
    def _():
        out_ref[:] = jnp.zeros_like(out_ref)

    b = b_ref[:]
    hi = b >> 4
    lo = b & 15
    ih = lax.broadcasted_iota(jnp.int32, (GH, 1), 0)
    il = lax.broadcasted_iota(jnp.int32, (GL, 1), 0)
    oh_hi = (hi == ih).astype(jnp.bfloat16)
    oh_lo = (lo == il).astype(jnp.bfloat16)
    out_ref[:] += lax.dot_general(
        oh_hi, oh_lo, (((1,), (1,)), ((), ())),
        preferred_element_type=jnp.float32)


def _tc_counts(brow):
    return pl.pallas_call(
        _counts_body,
        grid=(N_PAD // CNT_BLK,),
        in_specs=[pl.BlockSpec((1, CNT_BLK), lambda i: (0, i))],
        out_specs=pl.BlockSpec((GH, GL), lambda i: (0, 0)),
        out_shape=jax.ShapeDtypeStruct((GH, GL), jnp.float32),
    )(brow)


def _mlp_body(sums_ref, cnt_ref, w1_ref, b1_ref, a_ref, w2_ref, b2_ref, out_ref):
    s = sums_ref[0] + sums_ref[1]
    emb = s[:NUM_GRAPHS] / jnp.clip(cnt_ref[:], 1.0, None)
    h = jnp.dot(emb, w1_ref[:], preferred_element_type=jnp.float32) + b1_ref[:]
    a = a_ref[0, 0]
    h = jnp.where(h >= 0, h, a * h)
    out_ref[:] = (
        jnp.dot(h, w2_ref[:], preferred_element_type=jnp.float32) + b2_ref[:]
    )


def kernel(x, batch, W1, b1, prelu_a, W2, b2):
    batch32 = batch.astype(jnp.int32)
    x3d = x.reshape(NFULL, CHUNK, D)
    batch2d = batch32.reshape(NFULL, CHUNK)
    zsum = jnp.zeros((ROWS, D), jnp.float32)
    brow = jnp.concatenate(
        [batch32, jnp.full((N_PAD - N_NODES,), 1 << 20, jnp.int32)]
    ).reshape(1, N_PAD)

    cnts = _tc_counts(brow)
    sums = _sc_segment_sum(x3d, batch2d, zsum)

    return pl.pallas_call(
        _mlp_body,
        out_shape=jax.ShapeDtypeStruct((NUM_GRAPHS, D), jnp.float32),
    )(sums, cnts.reshape(NUM_GRAPHS, 1), W1, b1.reshape(1, D),
      prelu_a.reshape(1, 1), W2, b2.reshape(1, D))


# speedup vs baseline: 1.2328x; 1.2328x over previous
import functools

import jax
import jax.numpy as jnp
from jax import lax
from jax.experimental import pallas as pl
from jax.experimental.pallas import tpu as pltpu
from jax.experimental.pallas import tpu_sc as plsc

N_NODES = 100000
D = 128
NUM_GRAPHS = 512

NC = 2
NS = 16
NW = NC * NS

CHUNK = 80
NFULL = N_NODES // CHUNK
TAIL = N_NODES - NFULL * CHUNK
ROWS = 640
RPT = ROWS // NS
NBUF = 6

GH = 32
GL = 16
CNT_BLK = 12800
N_PAD = 102400


def _sc_segment_sum(x3d, batch2d, xt, bt, zsum):
    mesh = plsc.VectorSubcoreMesh(core_axis_name="c", subcore_axis_name="s")

    @functools.partial(
        pl.kernel,
        out_type=jax.ShapeDtypeStruct((NC, ROWS, D), jnp.float32),
        mesh=mesh,
        scratch_types=(
            [pltpu.VMEM((CHUNK,), jnp.int32)] * NBUF
            + [pltpu.VMEM((CHUNK, D), jnp.float32)] * NBUF
            + [pltpu.VMEM_SHARED((ROWS, D), jnp.float32)]
            + [pltpu.SemaphoreType.DMA] * NBUF
        ),
    )
    def seg_sum(x_hbm, b_hbm, xt_hbm, bt_hbm, zsum_hbm, sums_out, *scr):
        cid = lax.axis_index("c")
        sid = lax.axis_index("s")
        wid = sid * NC + cid
        idx_bufs = scr[:NBUF]
        row_bufs = scr[NBUF:2 * NBUF]
        ssum = scr[2 * NBUF]
        sems = scr[2 * NBUF + 1:]

        pltpu.sync_copy(zsum_hbm.at[pl.ds(RPT * sid, RPT)],
                        ssum.at[pl.ds(RPT * sid, RPT)])
        plsc.subcore_barrier()

        nc_mine = jnp.where(wid < NFULL % NW, NFULL // NW + 1, NFULL // NW)

        def do_slot(j, b):
            c = wid + NW * j
            pltpu.make_async_copy(b_hbm.at[c], idx_bufs[b], sems[b]).wait()
            pltpu.make_async_copy(x_hbm.at[c], row_bufs[b], sems[b]).wait()
            pltpu.sync_copy(row_bufs[b], ssum.at[idx_bufs[b]], add=True)

            @pl.when(j + NBUF < nc_mine)
            def _():
                c2 = wid + NW * (j + NBUF)
                pltpu.async_copy(b_hbm.at[c2], idx_bufs[b], sems[b])
                pltpu.async_copy(x_hbm.at[c2], row_bufs[b], sems[b])

        for b in range(NBUF):
            pltpu.async_copy(b_hbm.at[wid + NW * b], idx_bufs[b], sems[b])
            pltpu.async_copy(x_hbm.at[wid + NW * b], row_bufs[b], sems[b])

        def body(g, carry):
            for b in range(NBUF):
                j = NBUF * g + b

                @pl.when(j < nc_mine)
                def _():
                    do_slot(j, b)

            return carry

        lax.fori_loop(0, (NFULL // NW + NBUF) // NBUF, body, 0)

        if TAIL:
            @pl.when(wid == NW - 1)
            def _():
                pltpu.sync_copy(bt_hbm, idx_bufs[0].at[pl.ds(0, TAIL)])
                pltpu.sync_copy(xt_hbm, row_bufs[0].at[pl.ds(0, TAIL)])
                pltpu.sync_copy(row_bufs[0].at[pl.ds(0, TAIL)],
                                ssum.at[idx_bufs[0].at[pl.ds(0, TAIL)]], add=True)

        plsc.subcore_barrier()

        pltpu.sync_copy(ssum.at[pl.ds(RPT * sid, RPT)],
                        sums_out.at[cid, pl.ds(RPT * sid, RPT)])

    return seg_sum(x3d, batch2d, xt, bt, zsum)


def _counts_body(b_ref, out_ref):
    i = pl.program_id(0)

    @pl.when(i == 0)
    def _():
        out_ref[:] = jnp.zeros_like(out_ref)

    b = b_ref[:]
    hi = b >> 4
    lo = b & 15
    ih = lax.broadcasted_iota(jnp.int32, (GH, 1), 0)
    il = lax.broadcasted_iota(jnp.int32, (GL, 1), 0)
    oh_hi = (hi == ih).astype(jnp.bfloat16)
    oh_lo = (lo == il).astype(jnp.bfloat16)
    out_ref[:] += lax.dot_general(
        oh_hi, oh_lo, (((1,), (1,)), ((), ())),
        preferred_element_type=jnp.float32)


def _tc_counts(brow):
    return pl.pallas_call(
        _counts_body,
        grid=(N_PAD // CNT_BLK,),
        in_specs=[pl.BlockSpec((1, CNT_BLK), lambda i: (0, i))],
        out_specs=pl.BlockSpec((GH, GL), lambda i: (0, 0)),
        out_shape=jax.ShapeDtypeStruct((GH, GL), jnp.float32),
    )(brow)


def _mlp_body(sums_ref, cnt_ref, w1_ref, b1_ref, a_ref, w2_ref, b2_ref, out_ref):
    s = sums_ref[0] + sums_ref[1]
    emb = s[:NUM_GRAPHS] / jnp.clip(cnt_ref[:], 1.0, None)
    h = jnp.dot(emb, w1_ref[:], preferred_element_type=jnp.float32) + b1_ref[:]
    a = a_ref[0, 0]
    h = jnp.where(h >= 0, h, a * h)
    out_ref[:] = (
        jnp.dot(h, w2_ref[:], preferred_element_type=jnp.float32) + b2_ref[:]
    )


def kernel(x, batch, W1, b1, prelu_a, W2, b2):
    batch32 = batch.astype(jnp.int32)
    nfull = NFULL * CHUNK
    x3d = x[:nfull].reshape(NFULL, CHUNK, D)
    batch2d = batch32[:nfull].reshape(NFULL, CHUNK)
    xt = x[nfull:] if TAIL else x[:8]
    bt = batch32[nfull:] if TAIL else batch32[:8]
    zsum = jnp.zeros((ROWS, D), jnp.float32)
    brow = jnp.concatenate(
        [batch32, jnp.full((N_PAD - N_NODES,), 1 << 20, jnp.int32)]
    ).reshape(1, N_PAD)

    cnts = _tc_counts(brow)
    sums = _sc_segment_sum(x3d, batch2d, xt, bt, zsum)

    return pl.pallas_call(
        _mlp_body,
        out_shape=jax.ShapeDtypeStruct((NUM_GRAPHS, D), jnp.float32),
    )(sums, cnts.reshape(NUM_GRAPHS, 1), W1, b1.reshape(1, D),
      prelu_a.reshape(1, 1), W2, b2.reshape(1, D))


# speedup vs baseline: 1.2717x; 1.0316x over previous
import functools

import jax
import jax.numpy as jnp
from jax import lax
from jax.experimental import pallas as pl
from jax.experimental.pallas import tpu as pltpu
from jax.experimental.pallas import tpu_sc as plsc

N_NODES = 100000
D = 128
NUM_GRAPHS = 512

NC = 2
NS = 16
NW = NC * NS

CHUNK = 80
NFULL = N_NODES // CHUNK
TAIL = N_NODES - NFULL * CHUNK
ROWS = 640
RPT = ROWS // NS
NBUF = 4

GH = 32
GL = 16
CNT_BLK = 12800
N_PAD = 102400


def _sc_segment_sum(x3d, batch2d, xt, bt, zsum):
    mesh = plsc.VectorSubcoreMesh(core_axis_name="c", subcore_axis_name="s")

    @functools.partial(
        pl.kernel,
        out_type=jax.ShapeDtypeStruct((NC, ROWS, D), jnp.float32),
        mesh=mesh,
        scratch_types=(
            [pltpu.VMEM((CHUNK,), jnp.int32)] * NBUF
            + [pltpu.VMEM((CHUNK, D), jnp.float32)] * NBUF
            + [pltpu.VMEM_SHARED((ROWS, D), jnp.float32)]
            + [pltpu.SemaphoreType.DMA] * NBUF
        ),
    )
    def seg_sum(x_hbm, b_hbm, xt_hbm, bt_hbm, zsum_hbm, sums_out, *scr):
        cid = lax.axis_index("c")
        sid = lax.axis_index("s")
        wid = sid * NC + cid
        idx_bufs = scr[:NBUF]
        row_bufs = scr[NBUF:2 * NBUF]
        ssum = scr[2 * NBUF]
        sems = scr[2 * NBUF + 1:]

        pltpu.sync_copy(zsum_hbm.at[pl.ds(RPT * sid, RPT)],
                        ssum.at[pl.ds(RPT * sid, RPT)])
        plsc.subcore_barrier()

        nc_mine = jnp.where(wid < NFULL % NW, NFULL // NW + 1, NFULL // NW)

        def do_slot(j, b):
            c = wid + NW * j
            pltpu.make_async_copy(b_hbm.at[c], idx_bufs[b], sems[b]).wait()
            pltpu.make_async_copy(x_hbm.at[c], row_bufs[b], sems[b]).wait()
            pltpu.sync_copy(row_bufs[b], ssum.at[idx_bufs[b]], add=True)

            @pl.when(j + NBUF < nc_mine)
            def _():
                c2 = wid + NW * (j + NBUF)
                pltpu.async_copy(b_hbm.at[c2], idx_bufs[b], sems[b])
                pltpu.async_copy(x_hbm.at[c2], row_bufs[b], sems[b])

        for b in range(NBUF):
            pltpu.async_copy(b_hbm.at[wid + NW * b], idx_bufs[b], sems[b])
            pltpu.async_copy(x_hbm.at[wid + NW * b], row_bufs[b], sems[b])

        def body(g, carry):
            for b in range(NBUF):
                j = NBUF * g + b

                @pl.when(j < nc_mine)
                def _():
                    do_slot(j, b)

            return carry

        lax.fori_loop(0, (NFULL // NW + NBUF) // NBUF, body, 0)

        if TAIL:
            @pl.when(wid == NW - 1)
            def _():
                pltpu.sync_copy(bt_hbm, idx_bufs[0].at[pl.ds(0, TAIL)])
                pltpu.sync_copy(xt_hbm, row_bufs[0].at[pl.ds(0, TAIL)])
                pltpu.sync_copy(row_bufs[0].at[pl.ds(0, TAIL)],
                                ssum.at[idx_bufs[0].at[pl.ds(0, TAIL)]], add=True)

        plsc.subcore_barrier()

        pltpu.sync_copy(ssum.at[pl.ds(RPT * sid, RPT)],
                        sums_out.at[cid, pl.ds(RPT * sid, RPT)])

    return seg_sum(x3d, batch2d, xt, bt, zsum)


def _counts_body(b_ref, out_ref):
    i = pl.program_id(0)

    @pl.when(i == 0)
    def _():
        out_ref[:] = jnp.zeros_like(out_ref)

    b = b_ref[:]
    hi = b >> 4
    lo = b & 15
    ih = lax.broadcasted_iota(jnp.int32, (GH, 1), 0)
    il = lax.broadcasted_iota(jnp.int32, (GL, 1), 0)
    oh_hi = (hi == ih).astype(jnp.bfloat16)
    oh_lo = (lo == il).astype(jnp.bfloat16)
    out_ref[:] += lax.dot_general(
        oh_hi, oh_lo, (((1,), (1,)), ((), ())),
        preferred_element_type=jnp.float32)


def _tc_counts(brow):
    return pl.pallas_call(
        _counts_body,
        grid=(N_PAD // CNT_BLK,),
        in_specs=[pl.BlockSpec((1, CNT_BLK), lambda i: (0, i))],
        out_specs=pl.BlockSpec((GH, GL), lambda i: (0, 0)),
        out_shape=jax.ShapeDtypeStruct((GH, GL), jnp.float32),
    )(brow)


def _mlp_body(sums_ref, cnt_ref, w1_ref, b1_ref, a_ref, w2_ref, b2_ref, out_ref):
    s = sums_ref[0] + sums_ref[1]
    emb = s[:NUM_GRAPHS] / jnp.clip(cnt_ref[:], 1.0, None)
    h = jnp.dot(emb, w1_ref[:], preferred_element_type=jnp.float32) + b1_ref[:]
    a = a_ref[0, 0]
    h = jnp.where(h >= 0, h, a * h)
    out_ref[:] = (
        jnp.dot(h, w2_ref[:], preferred_element_type=jnp.float32) + b2_ref[:]
    )


def kernel(x, batch, W1, b1, prelu_a, W2, b2):
    batch32 = batch.astype(jnp.int32)
    nfull = NFULL * CHUNK
    x3d = x[:nfull].reshape(NFULL, CHUNK, D)
    batch2d = batch32[:nfull].reshape(NFULL, CHUNK)
    xt = x[nfull:] if TAIL else x[:8]
    bt = batch32[nfull:] if TAIL else batch32[:8]
    zsum = jnp.zeros((ROWS, D), jnp.float32)
    brow = jnp.concatenate(
        [batch32, jnp.full((N_PAD - N_NODES,), 1 << 20, jnp.int32)]
    ).reshape(1, N_PAD)

    cnts = _tc_counts(brow)
    sums = _sc_segment_sum(x3d, batch2d, xt, bt, zsum)

    return pl.pallas_call(
        _mlp_body,
        out_shape=jax.ShapeDtypeStruct((NUM_GRAPHS, D), jnp.float32),
    )(sums, cnts.reshape(NUM_GRAPHS, 1), W1, b1.reshape(1, D),
      prelu_a.reshape(1, 1), W2, b2.reshape(1, D))
